# Initial kernel scaffold; baseline (speedup 1.0000x reference)
import functools

import jax
import jax.numpy as jnp
from jax import lax
from jax.experimental import pallas as pl
from jax.experimental.pallas import tpu as pltpu
from jax.experimental.pallas import tpu_sc as plsc

N = 10000
E = 320000
D = 128

NC = 2
NS = 16
NW = NC * NS
EPW = E // NW
C = 125
NCH = EPW // C
RPT = N // NS
RCH = RPT // C


def _sc_aggregate(node_feat, src3, dst3, ef4):
  mesh = plsc.VectorSubcoreMesh(core_axis_name="c", subcore_axis_name="s")

  @functools.partial(
      pl.kernel,
      mesh=mesh,
      out_type=jax.ShapeDtypeStruct((NC, N, D), jnp.float32),
      scratch_types=[
          pltpu.VMEM((NCH, C), jnp.int32),
          pltpu.VMEM((NCH, C), jnp.int32),
          pltpu.VMEM((C, D), jnp.float32),
          pltpu.VMEM((C, D), jnp.float32),
          pltpu.VMEM_SHARED((N, D), jnp.float32),
          pltpu.SemaphoreType.DMA,
          pltpu.SemaphoreType.DMA,
      ],
  )
  def k(node_hbm, src_hbm, dst_hbm, ef_hbm, out_hbm, src_v, dst_v, g_v, e_v,
        acc, sem_g, sem_e):
    cid = lax.axis_index("c")
    sid = lax.axis_index("s")
    wid = sid * NC + cid

    def zrow(i, _):
      r = i // (D // 16)
      col = (i % (D // 16)) * 16
      g_v[r, pl.ds(col, 16)] = jnp.zeros((16,), jnp.float32)
      return 0
    lax.fori_loop(0, C * (D // 16), zrow, 0)
    base = sid * RPT
    for t in range(RCH):
      pltpu.sync_copy(g_v, acc.at[pl.ds(base + t * C, C)])
    plsc.subcore_barrier()

    pltpu.sync_copy(src_hbm.at[wid], src_v)
    pltpu.sync_copy(dst_hbm.at[wid], dst_v)

    def chunk(j, _):
      cp_g = pltpu.async_copy(node_hbm.at[src_v.at[j]], g_v, sem_g)
      cp_e = pltpu.async_copy(ef_hbm.at[wid, j], e_v, sem_e)
      cp_g.wait()
      cp_e.wait()

      def body(i, _):
        r = i // (D // 16)
        col = (i % (D // 16)) * 16
        v = g_v[r, pl.ds(col, 16)] + e_v[r, pl.ds(col, 16)]
        g_v[r, pl.ds(col, 16)] = jnp.maximum(v, 0.0)
        return 0
      lax.fori_loop(0, C * (D // 16), body, 0)

      pltpu.sync_copy(g_v, acc.at[dst_v.at[j]], add=True)
      return 0

    lax.fori_loop(0, NCH, chunk, 0)
    plsc.subcore_barrier()

    for t in range(RCH):
      rows = pl.ds(base + t * C, C)
      pltpu.sync_copy(acc.at[rows], g_v)
      pltpu.sync_copy(g_v, out_hbm.at[cid, rows])

  return k(node_feat, src3, dst3, ef4)


def _tc_body(node_ref, p_ref, w_ref, b_ref, gamma_ref, beta_ref, out_ref):
  rst = node_ref[...] + p_ref[0] + p_ref[1]
  h = lax.dot_general(rst, w_ref[...], (((1,), (1,)), ((), ())),
                      preferred_element_type=jnp.float32) + b_ref[...]
  h = jnp.where(h >= 0, h, 0.01 * h)
  mean = jnp.mean(h, axis=0, keepdims=True)
  var = jnp.mean((h - mean) * (h - mean), axis=0, keepdims=True)
  out_ref[...] = gamma_ref[...] * (h - mean) * lax.rsqrt(var + 1e-5) + beta_ref[...]


@jax.jit
def kernel(node_feat, edge_index, edge_feat, W, b, gamma, beta):
  src3 = edge_index[0].reshape(NW, NCH, C)
  dst3 = edge_index[1].reshape(NW, NCH, C)
  ef4 = edge_feat.reshape(NW, NCH, C, D)

  partials = _sc_aggregate(node_feat, src3, dst3, ef4)

  out = pl.pallas_call(
      _tc_body,
      out_shape=jax.ShapeDtypeStruct((N, D), jnp.float32),
  )(node_feat, partials, W, b.reshape(1, D), gamma.reshape(1, D),
    beta.reshape(1, D))
  return out


# speedup vs baseline: 2.6047x; 2.6047x over previous
import functools

import jax
import jax.numpy as jnp
from jax import lax
from jax.experimental import pallas as pl
from jax.experimental.pallas import tpu as pltpu
from jax.experimental.pallas import tpu_sc as plsc

N = 10000
E = 320000
D = 128

NC = 2
NS = 16
NW = NC * NS
EPW = E // NW
C = 80
NCH = EPW // C
NB = 5
CPB = NCH // NB
Z = 80
NZ = N // Z
ZT = NZ // NS + 1


def _sc_aggregate(node_feat, src3, dst3, ef4):
  mesh = plsc.VectorSubcoreMesh(core_axis_name="c", subcore_axis_name="s")

  @functools.partial(
      pl.kernel,
      mesh=mesh,
      out_type=jax.ShapeDtypeStruct((NC, N, D), jnp.float32),
      scratch_types=[
          pltpu.VMEM((CPB, C), jnp.int32),
          pltpu.VMEM((CPB, C), jnp.int32),
          pltpu.VMEM((C, D), jnp.float32),
          pltpu.VMEM((C, D), jnp.float32),
          pltpu.VMEM_SHARED((N, D), jnp.float32),
          pltpu.SemaphoreType.DMA,
          pltpu.SemaphoreType.DMA,
      ],
  )
  def k(node_hbm, src_hbm, dst_hbm, ef_hbm, out_hbm, src_v, dst_v, g_v, e_v,
        acc, sem_g, sem_e):
    cid = lax.axis_index("c")
    sid = lax.axis_index("s")
    wid = sid * NC + cid

    def zrow(i, _):
      r = i // (D // 16)
      col = (i % (D // 16)) * 16
      g_v[r, pl.ds(col, 16)] = jnp.zeros((16,), jnp.float32)
      return 0
    lax.fori_loop(0, Z * (D // 16), zrow, 0)
    for t in range(ZT):
      z = sid + t * NS
      @pl.when(z < NZ)
      def _():
        pltpu.sync_copy(g_v, acc.at[pl.ds(z * Z, Z)])
    plsc.subcore_barrier()

    for blk in range(NB):
      pltpu.sync_copy(src_hbm.at[wid, blk], src_v)
      pltpu.sync_copy(dst_hbm.at[wid, blk], dst_v)

      def chunk(jj, _):
        j = blk * CPB + jj
        cp_g = pltpu.async_copy(node_hbm.at[src_v.at[jj]], g_v, sem_g)
        cp_e = pltpu.async_copy(ef_hbm.at[wid, j], e_v, sem_e)
        cp_g.wait()
        cp_e.wait()

        def body(i, _):
          r = i // (D // 16)
          col = (i % (D // 16)) * 16
          v = g_v[r, pl.ds(col, 16)] + e_v[r, pl.ds(col, 16)]
          g_v[r, pl.ds(col, 16)] = jnp.maximum(v, 0.0)
          return 0
        lax.fori_loop(0, C * (D // 16), body, 0)

        pltpu.sync_copy(g_v, acc.at[dst_v.at[jj]], add=True)
        return 0

      lax.fori_loop(0, CPB, chunk, 0)
    plsc.subcore_barrier()

    for t in range(ZT):
      z = sid + t * NS
      @pl.when(z < NZ)
      def _():
        rows = pl.ds(z * Z, Z)
        pltpu.sync_copy(acc.at[rows], g_v)
        pltpu.sync_copy(g_v, out_hbm.at[cid, rows])

  return k(node_feat, src3, dst3, ef4)


def _tc_body(node_ref, p_ref, w_ref, b_ref, gamma_ref, beta_ref, out_ref):
  rst = node_ref[...] + p_ref[0] + p_ref[1]
  h = lax.dot_general(rst, w_ref[...], (((1,), (1,)), ((), ())),
                      preferred_element_type=jnp.float32) + b_ref[...]
  h = jnp.where(h >= 0, h, 0.01 * h)
  mean = jnp.mean(h, axis=0, keepdims=True)
  var = jnp.mean((h - mean) * (h - mean), axis=0, keepdims=True)
  out_ref[...] = gamma_ref[...] * (h - mean) * lax.rsqrt(var + 1e-5) + beta_ref[...]


@jax.jit
def kernel(node_feat, edge_index, edge_feat, W, b, gamma, beta):
  src3 = edge_index[0].reshape(NW, NB, CPB, C)
  dst3 = edge_index[1].reshape(NW, NB, CPB, C)
  ef4 = edge_feat.reshape(NW, NCH, C, D)

  partials = _sc_aggregate(node_feat, src3, dst3, ef4)

  out = pl.pallas_call(
      _tc_body,
      out_shape=jax.ShapeDtypeStruct((N, D), jnp.float32),
  )(node_feat, partials, W, b.reshape(1, D), gamma.reshape(1, D),
    beta.reshape(1, D))
  return out


# speedup vs baseline: 7.4403x; 2.8565x over previous
import functools

import jax
import jax.numpy as jnp
from jax import lax
from jax.experimental import pallas as pl
from jax.experimental.pallas import tpu as pltpu
from jax.experimental.pallas import tpu_sc as plsc

N = 10000
E = 320000
D = 128

NC = 2
NS = 16
NW = NC * NS
EPW = E // NW
C = 40
NCH = EPW // C
NB = 5
CPB = NCH // NB
Z = 40
NZ = N // Z
ZT = NZ // NS + 1


def _relu_add(g_v, e_v, m_v):
  @plsc.parallel_loop(0, C, unroll=2)
  def _(r):
    for k in range(D // 16):
      col = k * 16
      v = g_v[r, pl.ds(col, 16)] + e_v[r, pl.ds(col, 16)]
      m_v[r, pl.ds(col, 16)] = jnp.maximum(v, 0.0)


def _sc_aggregate(node_feat, src3, dst3, ef4):
  mesh = plsc.VectorSubcoreMesh(core_axis_name="c", subcore_axis_name="s")

  @functools.partial(
      pl.kernel,
      mesh=mesh,
      out_type=jax.ShapeDtypeStruct((NC, N, D), jnp.float32),
      scratch_types=[
          pltpu.VMEM((CPB, C), jnp.int32),
          pltpu.VMEM((CPB, C), jnp.int32),
          pltpu.VMEM((C, D), jnp.float32),
          pltpu.VMEM((C, D), jnp.float32),
          pltpu.VMEM((C, D), jnp.float32),
          pltpu.VMEM((C, D), jnp.float32),
          pltpu.VMEM((C, D), jnp.float32),
          pltpu.VMEM((C, D), jnp.float32),
          pltpu.VMEM_SHARED((N, D), jnp.float32),
          pltpu.SemaphoreType.DMA,
          pltpu.SemaphoreType.DMA,
          pltpu.SemaphoreType.DMA,
          pltpu.SemaphoreType.DMA,
          pltpu.SemaphoreType.DMA,
      ],
  )
  def k(node_hbm, src_hbm, dst_hbm, ef_hbm, out_hbm, src_v, dst_v,
        g0, g1, e0, e1, m0, m1, acc,
        sem_g0, sem_g1, sem_e0, sem_e1, sem_s):
    cid = lax.axis_index("c")
    sid = lax.axis_index("s")
    wid = sid * NC + cid
    g = (g0, g1)
    e = (e0, e1)
    m = (m0, m1)
    sem_g = (sem_g0, sem_g1)
    sem_e = (sem_e0, sem_e1)

    @plsc.parallel_loop(0, Z, unroll=2)
    def _(r):
      for k_ in range(D // 16):
        g0[r, pl.ds(k_ * 16, 16)] = jnp.zeros((16,), jnp.float32)
    for t in range(ZT):
      z = sid + t * NS
      @pl.when(z < NZ)
      def _():
        pltpu.sync_copy(g0, acc.at[pl.ds(z * Z, Z)])
    plsc.subcore_barrier()

    for blk in range(NB):
      pltpu.sync_copy(src_hbm.at[wid, blk], src_v)
      pltpu.sync_copy(dst_hbm.at[wid, blk], dst_v)

      for par in range(2):
        pltpu.async_copy(node_hbm.at[src_v.at[par]], g[par], sem_g[par])
        pltpu.async_copy(ef_hbm.at[wid, blk, par], e[par], sem_e[par])

      def pair(jj2, _):
        for par in range(2):
          j = jj2 * 2 + par
          pltpu.make_async_copy(node_hbm.at[src_v.at[j]], g[par],
                                sem_g[par]).wait()
          pltpu.make_async_copy(ef_hbm.at[wid, blk, j], e[par],
                                sem_e[par]).wait()
          @pl.when(jj2 >= 1)
          def _():
            pltpu.make_async_copy(m[par], acc.at[dst_v.at[j - 2]], sem_s).wait()
          _relu_add(g[par], e[par], m[par])
          @pl.when(jj2 < CPB // 2 - 1)
          def _():
            pltpu.async_copy(node_hbm.at[src_v.at[j + 2]], g[par], sem_g[par])
            pltpu.async_copy(ef_hbm.at[wid, blk, j + 2], e[par], sem_e[par])
          pltpu.async_copy(m[par], acc.at[dst_v.at[j]], sem_s, add=True)
        return 0

      lax.fori_loop(0, CPB // 2, pair, 0)
      for par in range(2):
        pltpu.make_async_copy(m[par], acc.at[dst_v.at[CPB - 2 + par]],
                              sem_s).wait()
    plsc.subcore_barrier()

    for t in range(ZT):
      z = sid + t * NS
      @pl.when(z < NZ)
      def _():
        rows = pl.ds(z * Z, Z)
        pltpu.sync_copy(acc.at[rows], g0)
        pltpu.sync_copy(g0, out_hbm.at[cid, rows])

  return k(node_feat, src3, dst3, ef4)


def _tc_body(node_ref, p_ref, w_ref, b_ref, gamma_ref, beta_ref, out_ref):
  rst = node_ref[...] + p_ref[0] + p_ref[1]
  h = lax.dot_general(rst, w_ref[...], (((1,), (1,)), ((), ())),
                      preferred_element_type=jnp.float32) + b_ref[...]
  h = jnp.where(h >= 0, h, 0.01 * h)
  mean = jnp.mean(h, axis=0, keepdims=True)
  var = jnp.mean((h - mean) * (h - mean), axis=0, keepdims=True)
  out_ref[...] = gamma_ref[...] * (h - mean) * lax.rsqrt(var + 1e-5) + beta_ref[...]


@jax.jit
def kernel(node_feat, edge_index, edge_feat, W, b, gamma, beta):
  src3 = edge_index[0].reshape(NW, NB, CPB, C)
  dst3 = edge_index[1].reshape(NW, NB, CPB, C)
  ef4 = edge_feat.reshape(NW, NB, CPB, C, D)

  partials = _sc_aggregate(node_feat, src3, dst3, ef4)

  out = pl.pallas_call(
      _tc_body,
      out_shape=jax.ShapeDtypeStruct((N, D), jnp.float32),
  )(node_feat, partials, W, b.reshape(1, D), gamma.reshape(1, D),
    beta.reshape(1, D))
  return out


# speedup vs baseline: 7.5139x; 1.0099x over previous
import functools

import jax
import jax.numpy as jnp
from jax import lax
from jax.experimental import pallas as pl
from jax.experimental.pallas import tpu as pltpu
from jax.experimental.pallas import tpu_sc as plsc

N = 10000
E = 320000
D = 128

NC = 2
NS = 16
NW = NC * NS
EPW = E // NW
C = 40
NCH = EPW // C
NB = 5
CPB = NCH // NB
Z = 40
NZ = N // Z
ZT = NZ // NS + 1


def _relu_add(g_v, e_v, m_v):
  @plsc.parallel_loop(0, C, unroll=4)
  def _(r):
    for k in range(D // 16):
      col = k * 16
      v = g_v[r, pl.ds(col, 16)] + e_v[r, pl.ds(col, 16)]
      m_v[r, pl.ds(col, 16)] = jnp.maximum(v, 0.0)


def _sc_aggregate(node_feat, src3, dst3, ef4):
  mesh = plsc.VectorSubcoreMesh(core_axis_name="c", subcore_axis_name="s")

  @functools.partial(
      pl.kernel,
      mesh=mesh,
      out_type=jax.ShapeDtypeStruct((NC, N, D), jnp.float32),
      scratch_types=[
          pltpu.VMEM((CPB, C), jnp.int32),
          pltpu.VMEM((CPB, C), jnp.int32),
          pltpu.VMEM((C, D), jnp.float32),
          pltpu.VMEM((C, D), jnp.float32),
          pltpu.VMEM((C, D), jnp.float32),
          pltpu.VMEM((C, D), jnp.float32),
          pltpu.VMEM((C, D), jnp.float32),
          pltpu.VMEM((C, D), jnp.float32),
          pltpu.VMEM_SHARED((N, D), jnp.float32),
          pltpu.SemaphoreType.DMA,
          pltpu.SemaphoreType.DMA,
          pltpu.SemaphoreType.DMA,
          pltpu.SemaphoreType.DMA,
          pltpu.SemaphoreType.DMA,
      ],
  )
  def k(node_hbm, src_hbm, dst_hbm, ef_hbm, out_hbm, src_v, dst_v,
        g0, g1, e0, e1, m0, m1, acc,
        sem_g0, sem_g1, sem_e0, sem_e1, sem_s):
    cid = lax.axis_index("c")
    sid = lax.axis_index("s")
    wid = sid * NC + cid
    g = (g0, g1)
    e = (e0, e1)
    m = (m0, m1)
    sem_g = (sem_g0, sem_g1)
    sem_e = (sem_e0, sem_e1)

    @plsc.parallel_loop(0, Z, unroll=2)
    def _(r):
      for k_ in range(D // 16):
        g0[r, pl.ds(k_ * 16, 16)] = jnp.zeros((16,), jnp.float32)
    zsems = (sem_g0, sem_g1, sem_e0, sem_e1)
    for t in range(ZT):
      z = sid + t * NS
      @pl.when(z < NZ)
      def _():
        pltpu.async_copy(g0, acc.at[pl.ds(z * Z, Z)], zsems[t % 4])
    for t in range(ZT):
      z = sid + t * NS
      @pl.when(z < NZ)
      def _():
        pltpu.make_async_copy(g0, acc.at[pl.ds(z * Z, Z)], zsems[t % 4]).wait()
    plsc.subcore_barrier()

    for blk in range(NB):
      pltpu.sync_copy(src_hbm.at[wid, blk], src_v)
      pltpu.sync_copy(dst_hbm.at[wid, blk], dst_v)

      for par in range(2):
        pltpu.async_copy(node_hbm.at[src_v.at[par]], g[par], sem_g[par])
        pltpu.async_copy(ef_hbm.at[wid, blk, par], e[par], sem_e[par])

      def pair(jj2, _):
        for par in range(2):
          j = jj2 * 2 + par
          pltpu.make_async_copy(node_hbm.at[src_v.at[j]], g[par],
                                sem_g[par]).wait()
          pltpu.make_async_copy(ef_hbm.at[wid, blk, j], e[par],
                                sem_e[par]).wait()
          @pl.when(jj2 >= 1)
          def _():
            pltpu.make_async_copy(m[par], acc.at[dst_v.at[j - 2]], sem_s).wait()
          _relu_add(g[par], e[par], m[par])
          @pl.when(jj2 < CPB // 2 - 1)
          def _():
            pltpu.async_copy(node_hbm.at[src_v.at[j + 2]], g[par], sem_g[par])
            pltpu.async_copy(ef_hbm.at[wid, blk, j + 2], e[par], sem_e[par])
          pltpu.async_copy(m[par], acc.at[dst_v.at[j]], sem_s, add=True)
        return 0

      lax.fori_loop(0, CPB // 2, pair, 0)
      for par in range(2):
        pltpu.make_async_copy(m[par], acc.at[dst_v.at[CPB - 2 + par]],
                              sem_s).wait()
    plsc.subcore_barrier()

    for t in range(ZT):
      z = sid + t * NS
      @pl.when(z < NZ)
      def _():
        rows = pl.ds(z * Z, Z)
        pltpu.async_copy(acc.at[rows], out_hbm.at[cid, rows], zsems[t % 4])
    for t in range(ZT):
      z = sid + t * NS
      @pl.when(z < NZ)
      def _():
        rows = pl.ds(z * Z, Z)
        pltpu.make_async_copy(acc.at[rows], out_hbm.at[cid, rows],
                              zsems[t % 4]).wait()

  return k(node_feat, src3, dst3, ef4)


def _tc_body(node_ref, p_ref, w_ref, b_ref, gamma_ref, beta_ref, out_ref):
  rst = node_ref[...] + p_ref[0] + p_ref[1]
  h = lax.dot_general(rst, w_ref[...], (((1,), (1,)), ((), ())),
                      preferred_element_type=jnp.float32) + b_ref[...]
  h = jnp.where(h >= 0, h, 0.01 * h)
  mean = jnp.mean(h, axis=0, keepdims=True)
  var = jnp.mean((h - mean) * (h - mean), axis=0, keepdims=True)
  out_ref[...] = gamma_ref[...] * (h - mean) * lax.rsqrt(var + 1e-5) + beta_ref[...]


@jax.jit
def kernel(node_feat, edge_index, edge_feat, W, b, gamma, beta):
  src3 = edge_index[0].reshape(NW, NB, CPB, C)
  dst3 = edge_index[1].reshape(NW, NB, CPB, C)
  ef4 = edge_feat.reshape(NW, NB, CPB, C, D)

  partials = _sc_aggregate(node_feat, src3, dst3, ef4)

  out = pl.pallas_call(
      _tc_body,
      out_shape=jax.ShapeDtypeStruct((N, D), jnp.float32),
  )(node_feat, partials, W, b.reshape(1, D), gamma.reshape(1, D),
    beta.reshape(1, D))
  return out


# speedup vs baseline: 7.5175x; 1.0005x over previous
import functools

import jax
import jax.numpy as jnp
from jax import lax
from jax.experimental import pallas as pl
from jax.experimental.pallas import tpu as pltpu
from jax.experimental.pallas import tpu_sc as plsc

N = 10000
E = 320000
D = 128

NC = 2
NS = 16
NW = NC * NS
EPW = E // NW
C = 40
NCH = EPW // C
NB = 5
CPB = NCH // NB
Z = 40
NZ = N // Z
ZT = NZ // NS + 1


def _relu_add(g_v, e_v, m_v):
  @plsc.parallel_loop(0, C, unroll=4)
  def _(r):
    for k in range(D // 16):
      col = k * 16
      v = g_v[r, pl.ds(col, 16)] + e_v[r, pl.ds(col, 16)]
      m_v[r, pl.ds(col, 16)] = jnp.maximum(v, 0.0)


def _sc_aggregate(node_feat, src3, dst3, ef4):
  mesh = plsc.VectorSubcoreMesh(core_axis_name="c", subcore_axis_name="s")

  @functools.partial(
      pl.kernel,
      mesh=mesh,
      out_type=jax.ShapeDtypeStruct((NC, N, D), jnp.float32),
      scratch_types=[
          pltpu.VMEM((CPB, C), jnp.int32),
          pltpu.VMEM((CPB, C), jnp.int32),
          pltpu.VMEM((C, D), jnp.float32),
          pltpu.VMEM((C, D), jnp.float32),
          pltpu.VMEM((C, D), jnp.float32),
          pltpu.VMEM((C, D), jnp.float32),
          pltpu.VMEM((C, D), jnp.float32),
          pltpu.VMEM((C, D), jnp.float32),
          pltpu.VMEM_SHARED((N, D), jnp.float32),
          pltpu.SemaphoreType.DMA,
          pltpu.SemaphoreType.DMA,
          pltpu.SemaphoreType.DMA,
          pltpu.SemaphoreType.DMA,
          pltpu.SemaphoreType.DMA,
      ],
  )
  def k(node_hbm, src_hbm, dst_hbm, ef_hbm, out_hbm, src_v, dst_v,
        g0, g1, e0, e1, m0, m1, acc,
        sem_g0, sem_g1, sem_e0, sem_e1, sem_s):
    cid = lax.axis_index("c")
    sid = lax.axis_index("s")
    wid = sid * NC + cid
    g = (g0, g1)
    e = (e0, e1)
    m = (m0, m1)
    sem_g = (sem_g0, sem_g1)
    sem_e = (sem_e0, sem_e1)

    with jax.named_scope("ph_init"):
      @plsc.parallel_loop(0, Z, unroll=2)
      def _(r):
        for k_ in range(D // 16):
          g0[r, pl.ds(k_ * 16, 16)] = jnp.zeros((16,), jnp.float32)
      zsems = (sem_g0, sem_g1, sem_e0, sem_e1)
      for t in range(ZT):
        z = sid + t * NS
        @pl.when(z < NZ)
        def _():
          pltpu.async_copy(g0, acc.at[pl.ds(z * Z, Z)], zsems[t % 4])
      for t in range(ZT):
        z = sid + t * NS
        @pl.when(z < NZ)
        def _():
          pltpu.make_async_copy(g0, acc.at[pl.ds(z * Z, Z)], zsems[t % 4]).wait()
      plsc.subcore_barrier()

    for blk in range(NB):
      pltpu.sync_copy(src_hbm.at[wid, blk], src_v)
      pltpu.sync_copy(dst_hbm.at[wid, blk], dst_v)

      for par in range(2):
        pltpu.async_copy(node_hbm.at[src_v.at[par]], g[par], sem_g[par])
        pltpu.async_copy(ef_hbm.at[wid, blk, par], e[par], sem_e[par])

      def pair(jj2, _):
        for par in range(2):
          j = jj2 * 2 + par
          pltpu.make_async_copy(node_hbm.at[src_v.at[j]], g[par],
                                sem_g[par]).wait()
          pltpu.make_async_copy(ef_hbm.at[wid, blk, j], e[par],
                                sem_e[par]).wait()
          @pl.when(jj2 >= 1)
          def _():
            pltpu.make_async_copy(m[par], acc.at[dst_v.at[j - 2]], sem_s).wait()
          _relu_add(g[par], e[par], m[par])
          @pl.when(jj2 < CPB // 2 - 1)
          def _():
            pltpu.async_copy(node_hbm.at[src_v.at[j + 2]], g[par], sem_g[par])
            pltpu.async_copy(ef_hbm.at[wid, blk, j + 2], e[par], sem_e[par])
          pltpu.async_copy(m[par], acc.at[dst_v.at[j]], sem_s, add=True)
        return 0

      with jax.named_scope("ph_loop"):
        lax.fori_loop(0, CPB // 2, pair, 0)
      for par in range(2):
        pltpu.make_async_copy(m[par], acc.at[dst_v.at[CPB - 2 + par]],
                              sem_s).wait()
    with jax.named_scope("ph_bar"):
      plsc.subcore_barrier()

    with jax.named_scope("ph_wb"):
      for t in range(ZT):
        z = sid + t * NS
        @pl.when(z < NZ)
        def _():
          rows = pl.ds(z * Z, Z)
          pltpu.async_copy(acc.at[rows], out_hbm.at[cid, rows], zsems[t % 4])
      for t in range(ZT):
        z = sid + t * NS
        @pl.when(z < NZ)
        def _():
          rows = pl.ds(z * Z, Z)
          pltpu.make_async_copy(acc.at[rows], out_hbm.at[cid, rows],
                                zsems[t % 4]).wait()

  return k(node_feat, src3, dst3, ef4)


def _tc_body(node_ref, p_ref, w_ref, b_ref, gamma_ref, beta_ref, out_ref):
  rst = node_ref[...] + p_ref[0] + p_ref[1]
  h = lax.dot_general(rst, w_ref[...], (((1,), (1,)), ((), ())),
                      preferred_element_type=jnp.float32) + b_ref[...]
  h = jnp.where(h >= 0, h, 0.01 * h)
  mean = jnp.mean(h, axis=0, keepdims=True)
  var = jnp.mean((h - mean) * (h - mean), axis=0, keepdims=True)
  out_ref[...] = gamma_ref[...] * (h - mean) * lax.rsqrt(var + 1e-5) + beta_ref[...]


@jax.jit
def kernel(node_feat, edge_index, edge_feat, W, b, gamma, beta):
  src3 = edge_index[0].reshape(NW, NB, CPB, C)
  dst3 = edge_index[1].reshape(NW, NB, CPB, C)
  ef4 = edge_feat.reshape(NW, NB, CPB, C, D)

  partials = _sc_aggregate(node_feat, src3, dst3, ef4)

  out = pl.pallas_call(
      _tc_body,
      out_shape=jax.ShapeDtypeStruct((N, D), jnp.float32),
  )(node_feat, partials, W, b.reshape(1, D), gamma.reshape(1, D),
    beta.reshape(1, D))
  return out
